# Initial kernel scaffold; baseline (speedup 1.0000x reference)
import jax
import jax.numpy as jnp
from jax.experimental import pallas as pl


def kernel(x, edge_index, edge_weight):
    raise NotImplementedError("write your pallas kernel here")


# speedup vs baseline: 4.0341x; 4.0341x over previous
import functools

import jax
import jax.numpy as jnp
from jax import lax
from jax.experimental import pallas as pl
from jax.experimental.pallas import tpu as pltpu
from jax.experimental.pallas import tpu_sc as plsc

N = 10000
E = 320000
D = 128
K = 3

NC = 2
NS = 16
L = 16

CHUNK = 128
N_PAD = 10240
ROWS_PER_TILE = N_PAD // NS
NCH = -(-E // (NC * NS * CHUNK))
E_HALF = NS * NCH * CHUNK
E_PAD = NC * E_HALF

_mesh = plsc.VectorSubcoreMesh(core_axis_name="c", subcore_axis_name="s")


def _hop_body(with_deg, x_hbm, rows_hbm, cols_hbm, w_hbm, *rest):
  if with_deg:
    (out_hbm, degout_hbm, xbuf, rows_v, cols_v, w_v, vbuf,
     y_sh, deg_sh, sem) = rest
  else:
    out_hbm, xbuf, rows_v, cols_v, w_v, vbuf, y_sh, sem = rest
    degout_hbm = deg_sh = None
  cid = lax.axis_index("c")
  sid = lax.axis_index("s")
  row0 = sid * ROWS_PER_TILE

  def _zrow(i, _):
    for j in range(D // L):
      xbuf[i, pl.ds(j * L, L)] = jnp.zeros((L,), jnp.float32)
    return 0
  lax.fori_loop(0, CHUNK, _zrow, 0)
  for k in range(ROWS_PER_TILE // CHUNK):
    pltpu.sync_copy(xbuf, y_sh.at[pl.ds(row0 + k * CHUNK, CHUNK)])
  if with_deg:
    def _zv(i, _):
      w_v[pl.ds(i * L, L)] = jnp.zeros((L,), jnp.float32)
      return 0
    lax.fori_loop(0, CHUNK // L, _zv, 0)
    for k in range(ROWS_PER_TILE // CHUNK):
      pltpu.sync_copy(w_v, deg_sh.at[pl.ds(row0 + k * CHUNK, CHUNK)])
  plsc.subcore_barrier()

  ebase = cid * E_HALF + sid * (NCH * CHUNK)

  def _chunk(ch, _):
    off = ebase + ch * CHUNK
    pltpu.sync_copy(rows_hbm.at[pl.ds(off, CHUNK)], rows_v)
    pltpu.sync_copy(cols_hbm.at[pl.ds(off, CHUNK)], cols_v)
    pltpu.sync_copy(w_hbm.at[pl.ds(off, CHUNK)], w_v)
    pltpu.async_copy(x_hbm.at[cols_v], xbuf, sem).wait()
    for g in range(CHUNK // L):
      sl = pl.ds(g * L, L)
      r16 = rows_v[sl]
      c16 = cols_v[sl]
      w16 = w_v[sl]
      vbuf[sl] = jnp.where(r16 == c16, jnp.zeros((L,), jnp.float32), w16)
    def _scale(i, _):
      vb = plsc.load_gather(vbuf, [jnp.full((L,), i, jnp.int32)])
      for j in range(D // L):
        s = pl.ds(j * L, L)
        xbuf[i, s] = xbuf[i, s] * vb
      return 0
    lax.fori_loop(0, CHUNK, _scale, 0)
    pltpu.sync_copy(xbuf, y_sh.at[rows_v], add=True)
    if with_deg:
      pltpu.sync_copy(vbuf, deg_sh.at[rows_v], add=True)
    return 0
  lax.fori_loop(0, NCH, _chunk, 0)
  plsc.subcore_barrier()

  pltpu.sync_copy(y_sh.at[pl.ds(row0, ROWS_PER_TILE)],
                  out_hbm.at[cid, pl.ds(row0, ROWS_PER_TILE)])
  if with_deg:
    pltpu.sync_copy(deg_sh.at[pl.ds(row0, ROWS_PER_TILE)],
                    degout_hbm.at[cid, pl.ds(row0, ROWS_PER_TILE)])


def _sc_hop(x_cur, rows, cols, w, with_deg):
  out_type = [jax.ShapeDtypeStruct((NC, N_PAD, D), jnp.float32)]
  scratch = [
      pltpu.VMEM((CHUNK, D), jnp.float32),
      pltpu.VMEM((CHUNK,), jnp.int32),
      pltpu.VMEM((CHUNK,), jnp.int32),
      pltpu.VMEM((CHUNK,), jnp.float32),
      pltpu.VMEM((CHUNK,), jnp.float32),
      pltpu.VMEM_SHARED((N_PAD, D), jnp.float32),
  ]
  if with_deg:
    out_type.append(jax.ShapeDtypeStruct((NC, N_PAD), jnp.float32))
    scratch.append(pltpu.VMEM_SHARED((N_PAD,), jnp.float32))
  scratch.append(pltpu.SemaphoreType.DMA)
  fn = pl.kernel(
      functools.partial(_hop_body, with_deg),
      out_type=tuple(out_type),
      mesh=_mesh,
      scratch_types=scratch,
      name="sc_hop_deg" if with_deg else "sc_hop",
      compiler_params=pltpu.CompilerParams(needs_layout_passes=False),
  )
  res = fn(x_cur, rows, cols, w)
  return res if with_deg else res[0]


BN = 512


def _combine1_body(p_ref, x_ref, pdeg_ref, y_ref, dinv_ref):
  pd = pdeg_ref[0] + pdeg_ref[1]
  deg = 1.0 + pd
  dinv = jnp.where(deg == 0.0, 0.0, 1.0 / deg)
  y_ref[...] = (p_ref[0] + p_ref[1] + x_ref[...]) * dinv
  dinv_ref[...] = dinv


def _combine1(p, x, pdeg):
  return pl.pallas_call(
      _combine1_body,
      grid=(N_PAD // BN,),
      in_specs=[
          pl.BlockSpec((NC, BN, D), lambda i: (0, i, 0)),
          pl.BlockSpec((BN, D), lambda i: (i, 0)),
          pl.BlockSpec((NC, BN, 1), lambda i: (0, i, 0)),
      ],
      out_specs=[
          pl.BlockSpec((BN, D), lambda i: (i, 0)),
          pl.BlockSpec((BN, 1), lambda i: (i, 0)),
      ],
      out_shape=[
          jax.ShapeDtypeStruct((N_PAD, D), jnp.float32),
          jax.ShapeDtypeStruct((N_PAD, 1), jnp.float32),
      ],
  )(p, x, pdeg.reshape(NC, N_PAD, 1))


def _combineN_body(p_ref, x_ref, dinv_ref, y_ref):
  y_ref[...] = (p_ref[0] + p_ref[1] + x_ref[...]) * dinv_ref[...]


def _combineN(p, x_prev, dinv):
  return pl.pallas_call(
      _combineN_body,
      grid=(N_PAD // BN,),
      in_specs=[
          pl.BlockSpec((NC, BN, D), lambda i: (0, i, 0)),
          pl.BlockSpec((BN, D), lambda i: (i, 0)),
          pl.BlockSpec((BN, 1), lambda i: (i, 0)),
      ],
      out_specs=pl.BlockSpec((BN, D), lambda i: (i, 0)),
      out_shape=jax.ShapeDtypeStruct((N_PAD, D), jnp.float32),
  )(p, x_prev, dinv)


def _mean_body(x_ref, o_ref):
  i = pl.program_id(0)
  s = jnp.sum(x_ref[...], axis=0, keepdims=True) * (1.0 / N)

  @pl.when(i == 0)
  def _():
    o_ref[...] = s

  @pl.when(i > 0)
  def _():
    o_ref[...] += s


def _mean(x):
  return pl.pallas_call(
      _mean_body,
      grid=(N_PAD // BN,),
      in_specs=[pl.BlockSpec((BN, D), lambda i: (i, 0))],
      out_specs=pl.BlockSpec((1, D), lambda i: (0, 0)),
      out_shape=jax.ShapeDtypeStruct((1, D), jnp.float32),
  )(x)


def _direction(x, rows, cols, w):
  outs = []
  p, pdeg = _sc_hop(x, rows, cols, w, with_deg=True)
  y, dinv = _combine1(p, x, pdeg)
  outs.append(y)
  for _ in range(K - 1):
    p = _sc_hop(y, rows, cols, w, with_deg=False)
    y = _combineN(p, y, dinv)
    outs.append(y)
  return outs


def kernel(x, edge_index, edge_weight):
  x = x.astype(jnp.float32)
  ei0 = edge_index[0]
  ei1 = edge_index[1]
  pad = E_PAD - E
  zi = jnp.zeros((pad,), jnp.int32)
  zf = jnp.zeros((pad,), jnp.float32)
  w_pad = jnp.concatenate([edge_weight.astype(jnp.float32), zf])
  rows_f = jnp.concatenate([ei1, zi])
  cols_f = jnp.concatenate([ei0, zi])
  rows_b = jnp.concatenate([ei0, zi])
  cols_b = jnp.concatenate([ei1, zi])

  x_pad = jnp.pad(x, ((0, N_PAD - N), (0, 0)))
  outs = [x_pad]
  outs += _direction(x_pad, rows_f, cols_f, w_pad)
  outs += _direction(x_pad, rows_b, cols_b, w_pad)
  g = _mean(x_pad)
  outs.append(jnp.broadcast_to(g, (N_PAD, D)))
  return jnp.concatenate(outs, axis=-1)[:N]
